# baseline (speedup 1.0000x reference)
import functools

import jax
import jax.numpy as jnp
from jax import lax
from jax.experimental import pallas as pl
from jax.experimental.pallas import tpu as pltpu
from jax.experimental.pallas import tpu_sc as plsc

FEATURES = 32
BATCH = 4096
HIST = 50
TOTAL = BATCH * HIST
NUM_WORKERS = 32
CHUNK = 128
ROWS_PER_W = TOTAL // NUM_WORKERS
CHUNKS_PER_W = ROWS_PER_W // CHUNK


def _build():
  mesh = plsc.VectorSubcoreMesh(core_axis_name="c", subcore_axis_name="s")

  @functools.partial(
      pl.kernel,
      mesh=mesh,
      compiler_params=pltpu.CompilerParams(use_tc_tiling_on_sc=False),
      out_type=jax.ShapeDtypeStruct((TOTAL, FEATURES), jnp.float32),
      scratch_types=[
          pltpu.VMEM((CHUNKS_PER_W, CHUNK), jnp.int32),
          pltpu.VMEM((CHUNK, FEATURES), jnp.float32),
          pltpu.SemaphoreType.DMA,
      ],
  )
  def k(idx_hbm, table_hbm, out_hbm, idx_v, rows_v, sem):
    wid = lax.axis_index("s") * 2 + lax.axis_index("c")
    pltpu.sync_copy(idx_hbm.at[wid], idx_v)

    def body(j, carry):
      pltpu.async_copy(table_hbm.at[idx_v.at[j]], rows_v, sem).wait()
      pltpu.sync_copy(
          rows_v, out_hbm.at[pl.ds((wid * CHUNKS_PER_W + j) * CHUNK, CHUNK)]
      )
      return carry

    lax.fori_loop(0, CHUNKS_PER_W, body, 0)

  return k


_gather_kernel = _build()


def kernel(inputs, embedding):
  idx = inputs.reshape(NUM_WORKERS, CHUNKS_PER_W, CHUNK).astype(jnp.int32)
  out = _gather_kernel(idx, embedding)
  return out.reshape(BATCH, HIST, FEATURES)


# speedup vs baseline: 1.0005x; 1.0005x over previous
import functools

import jax
import jax.numpy as jnp
from jax import lax
from jax.experimental import pallas as pl
from jax.experimental.pallas import tpu as pltpu
from jax.experimental.pallas import tpu_sc as plsc

FEATURES = 32
BATCH = 4096
HIST = 50
TOTAL = BATCH * HIST
NUM_WORKERS = 32
CHUNK = 128
ROWS_PER_W = TOTAL // NUM_WORKERS
CHUNKS_PER_W = ROWS_PER_W // CHUNK


def _build():
  mesh = plsc.VectorSubcoreMesh(core_axis_name="c", subcore_axis_name="s")

  @functools.partial(
      pl.kernel,
      mesh=mesh,
      compiler_params=pltpu.CompilerParams(
          use_tc_tiling_on_sc=False,
          skip_device_barrier=True,
          disable_semaphore_checks=True,
          disable_bounds_checks=True,
      ),
      out_type=jax.ShapeDtypeStruct((TOTAL, FEATURES), jnp.float32),
      scratch_types=[
          pltpu.VMEM((CHUNKS_PER_W, CHUNK), jnp.int32),
          pltpu.VMEM((CHUNK, FEATURES), jnp.float32),
          pltpu.SemaphoreType.DMA,
      ],
  )
  def k(idx_hbm, table_hbm, out_hbm, idx_v, rows_v, sem):
    wid = lax.axis_index("s") * 2 + lax.axis_index("c")
    pltpu.sync_copy(idx_hbm.at[wid], idx_v)

    def body(j, carry):
      pltpu.async_copy(table_hbm.at[idx_v.at[j]], rows_v, sem).wait()
      pltpu.sync_copy(
          rows_v, out_hbm.at[pl.ds((wid * CHUNKS_PER_W + j) * CHUNK, CHUNK)]
      )
      return carry

    lax.fori_loop(0, CHUNKS_PER_W, body, 0)

  return k


_gather_kernel = _build()


def kernel(inputs, embedding):
  idx = inputs.reshape(NUM_WORKERS, CHUNKS_PER_W, CHUNK).astype(jnp.int32)
  out = _gather_kernel(idx, embedding)
  return out.reshape(BATCH, HIST, FEATURES)


# speedup vs baseline: 1.1538x; 1.1533x over previous
import functools

import jax
import jax.numpy as jnp
from jax import lax
from jax.experimental import pallas as pl
from jax.experimental.pallas import tpu as pltpu
from jax.experimental.pallas import tpu_sc as plsc

FEATURES = 32
BATCH = 4096
HIST = 50
NUM_WORKERS = 32
BBLK = BATCH // NUM_WORKERS
NROWS = 1000000
L = 16
NBUF = 2


def _build():
  mesh = plsc.VectorSubcoreMesh(core_axis_name="c", subcore_axis_name="s")

  @functools.partial(
      pl.kernel,
      mesh=mesh,
      compiler_params=pltpu.CompilerParams(
          use_tc_tiling_on_sc=False, needs_layout_passes=False),
      out_type=jax.ShapeDtypeStruct((HIST, FEATURES, BATCH), jnp.float32),
      scratch_types=[
          pltpu.VMEM((HIST, BBLK), jnp.int32),
          [pltpu.VMEM((BBLK,), jnp.int32) for _ in range(NBUF)],
          [pltpu.VMEM((BBLK, 128), jnp.float32) for _ in range(NBUF)],
          [pltpu.VMEM((FEATURES, BBLK), jnp.float32) for _ in range(NBUF)],
          [pltpu.SemaphoreType.DMA for _ in range(NBUF)],
      ],
  )
  def k(idx_hbm, table_hbm, out_hbm, idx_v, rowid_v, buf_v, obuf_v, gsem):
    wid = lax.axis_index("s") * 2 + lax.axis_index("c")
    b0 = wid * BBLK
    pltpu.sync_copy(idx_hbm.at[:, pl.ds(b0, BBLK)], idx_v)

    iota = lax.iota(jnp.int32, L)

    def prep(h, slot):
      for g in range(BBLK // L):
        rowid_v[slot][pl.ds(g * L, L)] = idx_v[h, pl.ds(g * L, L)]
      pltpu.async_copy(
          table_hbm.at[rowid_v[slot]], buf_v[slot], gsem[slot])

    def step(h, slot):
      pltpu.make_async_copy(
          table_hbm.at[rowid_v[slot]], buf_v[slot], gsem[slot]).wait()
      for g in range(BBLK // L):
        rows = g * L + iota
        for c in range(FEATURES):
          obuf_v[slot][c, pl.ds(g * L, L)] = plsc.load_gather(
              buf_v[slot], [rows, jnp.full((L,), c, jnp.int32)])
      pltpu.sync_copy(obuf_v[slot], out_hbm.at[h, :, pl.ds(b0, BBLK)])
      prep(jnp.minimum(h + NBUF, HIST - 1), slot)

    for s in range(NBUF):
      prep(jnp.int32(s), s)

    def body(j, carry):
      for s in range(NBUF):
        step(j * NBUF + s, s)
      return carry

    lax.fori_loop(0, HIST // NBUF, body, jnp.int32(0))

    for s in range(NBUF):
      pltpu.make_async_copy(
          table_hbm.at[rowid_v[s]], buf_v[s], gsem[s]).wait()

  return k


_gather_kernel = _build()


def kernel(inputs, embedding):
  idx_t = inputs.T.astype(jnp.int32)
  table_pad = jnp.pad(embedding, ((0, 0), (0, 128 - FEATURES)))
  out_t = _gather_kernel(idx_t, table_pad)
  return out_t.transpose(2, 0, 1)


# speedup vs baseline: 1.3808x; 1.1967x over previous
import functools

import jax
import jax.numpy as jnp
from jax import lax
from jax.experimental import pallas as pl
from jax.experimental.pallas import tpu as pltpu
from jax.experimental.pallas import tpu_sc as plsc

FEATURES = 32
BATCH = 4096
HIST = 50
NUM_WORKERS = 32
BBLK = BATCH // NUM_WORKERS
NROWS = 1000000
L = 16
NBUF = 2


def _build():
  mesh = plsc.VectorSubcoreMesh(core_axis_name="c", subcore_axis_name="s")

  @functools.partial(
      pl.kernel,
      mesh=mesh,
      compiler_params=pltpu.CompilerParams(
          use_tc_tiling_on_sc=False, needs_layout_passes=False),
      out_type=jax.ShapeDtypeStruct((HIST, FEATURES, BATCH), jnp.float32),
      scratch_types=[
          pltpu.VMEM((HIST, BBLK), jnp.int32),
          [pltpu.VMEM((BBLK,), jnp.int32) for _ in range(NBUF)],
          [pltpu.VMEM((BBLK, 128), jnp.float32) for _ in range(NBUF)],
          [pltpu.VMEM((FEATURES, BBLK), jnp.float32) for _ in range(NBUF)],
          [pltpu.SemaphoreType.DMA for _ in range(NBUF)],
      ],
  )
  def k(idx_hbm, table_hbm, out_hbm, idx_v, rowid_v, buf_v, obuf_v, gsem):
    wid = lax.axis_index("s") * 2 + lax.axis_index("c")
    b0 = wid * BBLK
    pltpu.sync_copy(idx_hbm.at[:, pl.ds(b0, BBLK)], idx_v)

    iota = lax.iota(jnp.int32, L)

    def prep(h, slot):
      for g in range(BBLK // L):
        rowid_v[slot][pl.ds(g * L, L)] = idx_v[h, pl.ds(g * L, L)]
      pltpu.async_copy(
          table_hbm.at[rowid_v[slot]], buf_v[slot], gsem[slot])

    def step(h, slot):
      pltpu.make_async_copy(
          table_hbm.at[rowid_v[slot]], buf_v[slot], gsem[slot]).wait()
      def dbody(d, carry):
        colbase = lax.bitwise_and(iota + d, jnp.full((L,), L - 1, jnp.int32))
        for g in range(BBLK // L):
          rows = g * L + iota
          for c0 in range(0, FEATURES, L):
            cols = colbase + c0
            v = plsc.load_gather(buf_v[slot], [rows, cols])
            plsc.store_scatter(obuf_v[slot], [cols, rows], v)
        return carry

      lax.fori_loop(0, L, dbody, jnp.int32(0))
      pltpu.sync_copy(obuf_v[slot], out_hbm.at[h, :, pl.ds(b0, BBLK)])
      prep(jnp.minimum(h + NBUF, HIST - 1), slot)

    for s in range(NBUF):
      prep(jnp.int32(s), s)

    def body(j, carry):
      for s in range(NBUF):
        step(j * NBUF + s, s)
      return carry

    lax.fori_loop(0, HIST // NBUF, body, jnp.int32(0))

    for s in range(NBUF):
      pltpu.make_async_copy(
          table_hbm.at[rowid_v[s]], buf_v[s], gsem[s]).wait()

  return k


_gather_kernel = _build()


def kernel(inputs, embedding):
  idx_t = inputs.T.astype(jnp.int32)
  table_pad = jnp.pad(embedding, ((0, 0), (0, 128 - FEATURES)))
  out_t = _gather_kernel(idx_t, table_pad)
  return out_t.transpose(2, 0, 1)
